# baseline (speedup 1.0000x reference)
import functools

import jax
import jax.numpy as jnp
from jax import lax
from jax.experimental import pallas as pl
from jax.experimental.pallas import tpu as pltpu


_IMGS_PER_STEP = 4


def _ph_kernel(G, C, H, W, sW, x_ref, w_ref, b_ref, o_ref):
    r_w = (W - 1) / (sW - 1)
    win = lax.broadcasted_iota(jnp.int32, (W, sW), 0).astype(jnp.float32)
    wout = lax.broadcasted_iota(jnp.int32, (W, sW), 1).astype(jnp.float32)
    src_w = jnp.minimum(wout * r_w, W - 1)
    mwt = jnp.maximum(0.0, 1.0 - jnp.abs(src_w - win))

    sH = 2 * H
    r_h = (H - 1) / (sH - 1)
    hi = lax.broadcasted_iota(jnp.int32, (sH, H), 0).astype(jnp.float32)
    hk = lax.broadcasted_iota(jnp.int32, (sH, H), 1).astype(jnp.float32)
    src_h = jnp.minimum(hi * r_h, H - 1)
    a_h = jnp.maximum(0.0, 1.0 - jnp.abs(src_h - hk))

    bias = b_ref[0]
    for g in range(G):
        acc = x_ref[g, 0].astype(jnp.float32) * w_ref[0] + bias
        for c in range(1, C):
            acc = acc + x_ref[g, c].astype(jnp.float32) * w_ref[c]

        mid = jnp.dot(acc, mwt, preferred_element_type=jnp.float32)
        o_ref[g, 0] = jnp.dot(a_h, mid,
                              preferred_element_type=jnp.float32
                              ).astype(o_ref.dtype)


def kernel(w, b, x):
    B, C, H, W = x.shape
    sH, sW = 2 * H, 2 * W
    G = _IMGS_PER_STEP if B % _IMGS_PER_STEP == 0 else 1
    wv = w.astype(jnp.float32).reshape(C)
    bv = b.astype(jnp.float32).reshape(1)
    smem = pl.BlockSpec(memory_space=pltpu.MemorySpace.SMEM)
    return pl.pallas_call(
        functools.partial(_ph_kernel, G, C, H, W, sW),
        out_shape=jax.ShapeDtypeStruct((B, 1, sH, sW), jnp.float32),
        grid=(B // G,),
        in_specs=[pl.BlockSpec((G, C, H, W), lambda i: (i, 0, 0, 0)),
                  smem, smem],
        out_specs=pl.BlockSpec((G, 1, sH, sW), lambda i: (i, 0, 0, 0)),
        compiler_params=pltpu.CompilerParams(
            dimension_semantics=("parallel",),
            vmem_limit_bytes=96 * 1024 * 1024),
    )(x, wv, bv)


# speedup vs baseline: 1.0483x; 1.0483x over previous
import functools

import jax
import jax.numpy as jnp
from jax import lax
from jax.experimental import pallas as pl
from jax.experimental.pallas import tpu as pltpu


_IMGS_PER_STEP = 8


def _ph_kernel(G, C, H, W, sW, x_ref, w_ref, b_ref, o_ref):
    r_w = (W - 1) / (sW - 1)
    win = lax.broadcasted_iota(jnp.int32, (W, sW), 0).astype(jnp.float32)
    wout = lax.broadcasted_iota(jnp.int32, (W, sW), 1).astype(jnp.float32)
    src_w = jnp.minimum(wout * r_w, W - 1)
    mwt = jnp.maximum(0.0, 1.0 - jnp.abs(src_w - win))

    sH = 2 * H
    r_h = (H - 1) / (sH - 1)
    hi = lax.broadcasted_iota(jnp.int32, (sH, H), 0).astype(jnp.float32)
    hk = lax.broadcasted_iota(jnp.int32, (sH, H), 1).astype(jnp.float32)
    src_h = jnp.minimum(hi * r_h, H - 1)
    a_h = jnp.maximum(0.0, 1.0 - jnp.abs(src_h - hk))

    bias = b_ref[0]
    for g in range(G):
        acc = x_ref[g, 0].astype(jnp.float32) * w_ref[0] + bias
        for c in range(1, C):
            acc = acc + x_ref[g, c].astype(jnp.float32) * w_ref[c]

        mid = jnp.dot(acc, mwt, preferred_element_type=jnp.float32)
        o_ref[g, 0] = jnp.dot(a_h, mid,
                              preferred_element_type=jnp.float32
                              ).astype(o_ref.dtype)


def kernel(w, b, x):
    B, C, H, W = x.shape
    sH, sW = 2 * H, 2 * W
    G = _IMGS_PER_STEP if B % _IMGS_PER_STEP == 0 else 1
    wv = w.astype(jnp.float32).reshape(C)
    bv = b.astype(jnp.float32).reshape(1)
    smem = pl.BlockSpec(memory_space=pltpu.MemorySpace.SMEM)
    return pl.pallas_call(
        functools.partial(_ph_kernel, G, C, H, W, sW),
        out_shape=jax.ShapeDtypeStruct((B, 1, sH, sW), jnp.float32),
        grid=(B // G,),
        in_specs=[pl.BlockSpec((G, C, H, W), lambda i: (i, 0, 0, 0)),
                  smem, smem],
        out_specs=pl.BlockSpec((G, 1, sH, sW), lambda i: (i, 0, 0, 0)),
        compiler_params=pltpu.CompilerParams(
            dimension_semantics=("parallel",),
            vmem_limit_bytes=64 * 1024 * 1024),
    )(x, wv, bv)
